# baseline (speedup 1.0000x reference)
import functools

import jax
import jax.numpy as jnp
from jax import lax
from jax.experimental import pallas as pl
from jax.experimental.pallas import tpu as pltpu
from jax.experimental.pallas import tpu_sc as plsc

N = 10000
E = 320000
EL = 100000
H = 128
NW = 32
C = 80
EPW = 10240
EP = NW * EPW
NCH = EPW // C
IB = 16
NB = NCH // IB
NP = 10240
ROWS_PW = NP // 16
DC = 80
DPW = 102400 // NW
DNCH = DPW // DC
DIB = 8
SCP = pltpu.CompilerParams(needs_layout_passes=False, use_tc_tiling_on_sc=False)


def _prep_body(x_ref, emb_ref, lw_ref, lb_ref, w_ref, w2_ref, atta_ref,
               attb_ref, tab_ref, a1_ref, a2_ref):
    h = x_ref[...] * lw_ref[...] + lb_ref[...] + emb_ref[...]
    tab = jnp.dot(h, w_ref[...], preferred_element_type=jnp.float32)
    hd = jnp.dot(h, w2_ref[...], preferred_element_type=jnp.float32)
    tab_ref[...] = tab
    a1_ref[...] = jnp.sum(tab * atta_ref[...], axis=1, keepdims=True)
    a2_ref[...] = jnp.sum(hd * attb_ref[...], axis=1, keepdims=True)


def _prep(x, emb, lin_w, lin_b, w, w2, atta, attb, block=2000):
    return pl.pallas_call(
        _prep_body,
        grid=(N // block,),
        in_specs=[
            pl.BlockSpec((block, 1), lambda i: (i, 0)),
            pl.BlockSpec((block, H), lambda i: (i, 0)),
            pl.BlockSpec((1, H), lambda i: (0, 0)),
            pl.BlockSpec((1, H), lambda i: (0, 0)),
            pl.BlockSpec((H, H), lambda i: (0, 0)),
            pl.BlockSpec((H, H), lambda i: (0, 0)),
            pl.BlockSpec((1, H), lambda i: (0, 0)),
            pl.BlockSpec((1, H), lambda i: (0, 0)),
        ],
        out_specs=[
            pl.BlockSpec((block, H), lambda i: (i, 0)),
            pl.BlockSpec((block, 1), lambda i: (i, 0)),
            pl.BlockSpec((block, 1), lambda i: (i, 0)),
        ],
        out_shape=[
            jax.ShapeDtypeStruct((N, H), jnp.float32),
            jax.ShapeDtypeStruct((N, 1), jnp.float32),
            jax.ShapeDtypeStruct((N, 1), jnp.float32),
        ],
    )(x, emb, lin_w, lin_b, w, w2, atta, attb)


def _mid_body(part_ref, den_ref, bias_ref, w_ref, ba_ref, w2_ref, atta_ref,
              attb_ref, tab_ref, a1_ref, a2_ref, *, relu):
    num = part_ref[0] + part_ref[1]
    den = den_ref[0] + den_ref[1]
    z = num / (den + 1e-16) + bias_ref[...]
    if relu:
        z = jnp.maximum(z, 0.0)
    tab = jnp.dot(z, w_ref[...],
                  preferred_element_type=jnp.float32) + ba_ref[...]
    hd = jnp.dot(z, w2_ref[...], preferred_element_type=jnp.float32)
    tab_ref[...] = tab
    a1_ref[...] = jnp.sum(tab * atta_ref[...], axis=1, keepdims=True)
    a2_ref[...] = jnp.sum(hd * attb_ref[...], axis=1, keepdims=True)


def _mid(part, den, bias, w, ba, w2, atta, attb, relu, block=2000):
    wout = w.shape[1]
    return pl.pallas_call(
        functools.partial(_mid_body, relu=relu),
        grid=(N // block,),
        in_specs=[
            pl.BlockSpec((2, block, H), lambda i: (0, i, 0)),
            pl.BlockSpec((2, block, 1), lambda i: (0, i, 0)),
            pl.BlockSpec((1, H), lambda i: (0, 0)),
            pl.BlockSpec((H, wout), lambda i: (0, 0)),
            pl.BlockSpec((1, wout), lambda i: (0, 0)),
            pl.BlockSpec((H, H), lambda i: (0, 0)),
            pl.BlockSpec((1, wout), lambda i: (0, 0)),
            pl.BlockSpec((1, H), lambda i: (0, 0)),
        ],
        out_specs=[
            pl.BlockSpec((block, wout), lambda i: (i, 0)),
            pl.BlockSpec((block, 1), lambda i: (i, 0)),
            pl.BlockSpec((block, 1), lambda i: (i, 0)),
        ],
        out_shape=[
            jax.ShapeDtypeStruct((N, wout), jnp.float32),
            jax.ShapeDtypeStruct((N, 1), jnp.float32),
            jax.ShapeDtypeStruct((N, 1), jnp.float32),
        ],
    )(part, den, bias, w, ba, w2, atta, attb)


def _sc_gat_body(tab_hbm, asrc_hbm, adst_hbm, srcs_hbm, dsts_hbm,
                 out_hbm, dout_hbm,
                 srcs_v, dsts_v, asrc_v, adst_v, r0, r1, w0, w1,
                 acc_sh, den_sh, sg0, sg1, ss0, ss1):
    cid = lax.axis_index("c")
    sid = lax.axis_index("s")
    wid = cid * 16 + sid
    rows = (r0, r1)
    wcs = (w0, w1)
    sgs = (sg0, sg1)
    sss = (ss0, ss1)

    pltpu.sync_copy(asrc_hbm, asrc_v)
    pltpu.sync_copy(adst_hbm, adst_v)

    def zrow(j, carry):
        for k in range(H // 16):
            r0[j, pl.ds(k * 16, 16)] = jnp.zeros((16,), jnp.float32)
        return carry

    lax.fori_loop(0, C, zrow, 0)
    for q in range(C // 16):
        w0[pl.ds(q * 16, 16)] = jnp.zeros((16,), jnp.float32)
    base = sid * ROWS_PW
    for i in range(ROWS_PW // C):
        pltpu.sync_copy(r0, acc_sh.at[pl.ds(base + i * C, C)])
        pltpu.sync_copy(w0, den_sh.at[pl.ds(base + i * C, C)])
    plsc.subcore_barrier()

    def calc_w(i, p):
        for q in range(C // 16):
            s16 = srcs_v[i, pl.ds(q * 16, 16)]
            d16 = dsts_v[i, pl.ds(q * 16, 16)]
            t = plsc.load_gather(asrc_v, [s16]) + plsc.load_gather(adst_v, [d16])
            t = jnp.maximum(t, 0.2 * t)
            wcs[p][pl.ds(q * 16, 16)] = jnp.exp(t)

    def scale(p):
        buf = rows[p]
        wc = wcs[p]

        def srow(j4, c2):
            for u in range(4):
                j = j4 * 4 + u
                w = plsc.load_gather(wc, [jnp.full((16,), j, jnp.int32)])
                for k in range(H // 16):
                    buf[j, pl.ds(k * 16, 16)] = buf[j, pl.ds(k * 16, 16)] * w
            return c2

        lax.fori_loop(0, C // 4, srow, 0)

    def blk(b, carry):
        pltpu.sync_copy(srcs_hbm.at[wid, pl.ds(b * IB, IB)], srcs_v)
        pltpu.sync_copy(dsts_hbm.at[wid, pl.ds(b * IB, IB)], dsts_v)
        gets = {}
        puts = {}
        gets[0] = pltpu.async_copy(tab_hbm.at[srcs_v.at[0]], r0, sg0)
        for i in range(IB):
            p = i % 2
            if i + 1 < IB:
                if i + 1 >= 2:
                    for d in puts[i - 1]:
                        d.wait()
                gets[i + 1] = pltpu.async_copy(
                    tab_hbm.at[srcs_v.at[i + 1]], rows[1 - p], sgs[1 - p])
            calc_w(i, p)
            gets[i].wait()
            scale(p)
            puts[i] = (
                pltpu.async_copy(rows[p], acc_sh.at[dsts_v.at[i]], sss[p],
                                 add=True),
                pltpu.async_copy(wcs[p], den_sh.at[dsts_v.at[i]], sss[p],
                                 add=True),
            )
        for d in puts[IB - 2] + puts[IB - 1]:
            d.wait()
        return carry

    lax.fori_loop(0, NB, blk, 0)
    plsc.subcore_barrier()
    pltpu.sync_copy(acc_sh.at[pl.ds(base, ROWS_PW)],
                    out_hbm.at[cid, pl.ds(base, ROWS_PW)])
    pltpu.sync_copy(den_sh.at[pl.ds(base, ROWS_PW)],
                    dout_hbm.at[cid, pl.ds(base, ROWS_PW)])


def _sc_gat(tab, a_src, a_dst, srcs, dsts):
    mesh = plsc.VectorSubcoreMesh(core_axis_name="c", subcore_axis_name="s")
    return pl.kernel(
        _sc_gat_body,
        compiler_params=SCP,
        out_type=[
            jax.ShapeDtypeStruct((2, NP, H), jnp.float32),
            jax.ShapeDtypeStruct((2, NP), jnp.float32),
        ],
        mesh=mesh,
        scratch_types=[
            pltpu.VMEM((IB, C), jnp.int32),
            pltpu.VMEM((IB, C), jnp.int32),
            pltpu.VMEM((NP,), jnp.float32),
            pltpu.VMEM((NP,), jnp.float32),
            pltpu.VMEM((C, H), jnp.float32),
            pltpu.VMEM((C, H), jnp.float32),
            pltpu.VMEM((C,), jnp.float32),
            pltpu.VMEM((C,), jnp.float32),
            pltpu.VMEM_SHARED((NP, H), jnp.float32),
            pltpu.VMEM_SHARED((NP,), jnp.float32),
            pltpu.SemaphoreType.DMA,
            pltpu.SemaphoreType.DMA,
            pltpu.SemaphoreType.DMA,
            pltpu.SemaphoreType.DMA,
        ],
    )(tab, a_src, a_dst, srcs, dsts)


def _sc_dec_body(p_hbm, q_hbm, ridx_hbm, cidx_hbm, w2_hbm, out_hbm,
                 ridx_v, cidx_v, p0, p1, q0, q1, w2_v, tbuf, obuf,
                 sg0, sg1):
    cid = lax.axis_index("c")
    sid = lax.axis_index("s")
    wid = cid * 16 + sid
    pbufs = (p0, p1)
    qbufs = (q0, q1)
    sgs = (sg0, sg1)

    pltpu.sync_copy(ridx_hbm.at[wid], ridx_v)
    pltpu.sync_copy(cidx_hbm.at[wid], cidx_v)
    pltpu.sync_copy(w2_hbm, w2_v)
    lane = jnp.arange(16, dtype=jnp.int32)
    w2s = [w2_v[pl.ds(k * 16, 16)] for k in range(H // 16)]

    def compute(g, p):
        pb = pbufs[p]
        qb = qbufs[p]

        def grp(j16, c2):
            for jj in range(16):
                acc = jnp.zeros((16,), jnp.float32)
                for k in range(H // 16):
                    pv = pb[j16 * 16 + jj, pl.ds(k * 16, 16)]
                    qv = qb[j16 * 16 + jj, pl.ds(k * 16, 16)]
                    acc = acc + jnp.maximum(pv + qv, 0.0) * w2s[k]
                tbuf[jj, :] = acc
            res = jnp.zeros((16,), jnp.float32)
            for k in range(16):
                res = res + plsc.load_gather(
                    tbuf, [lane, jnp.full((16,), k, jnp.int32)])
            obuf[pl.ds(g * DC + j16 * 16, 16)] = res
            return c2

        lax.fori_loop(0, DC // 16, grp, 0)

    def blk(b, carry):
        g0 = b * DIB
        gets = {}
        gets[0] = (
            pltpu.async_copy(p_hbm.at[ridx_v.at[g0]], p0, sg0),
            pltpu.async_copy(q_hbm.at[cidx_v.at[g0]], q0, sg0),
        )
        for i in range(DIB):
            p = i % 2
            if i + 1 < DIB:
                gets[i + 1] = (
                    pltpu.async_copy(p_hbm.at[ridx_v.at[g0 + i + 1]],
                                     pbufs[1 - p], sgs[1 - p]),
                    pltpu.async_copy(q_hbm.at[cidx_v.at[g0 + i + 1]],
                                     qbufs[1 - p], sgs[1 - p]),
                )
            for d in gets[i]:
                d.wait()
            compute(g0 + i, p)
        return carry

    lax.fori_loop(0, DNCH // DIB, blk, 0)
    pltpu.sync_copy(obuf, out_hbm.at[pl.ds(wid * DPW, DPW)])


def _sc_dec(p, q, ridx, cidx, w2):
    mesh = plsc.VectorSubcoreMesh(core_axis_name="c", subcore_axis_name="s")
    return pl.kernel(
        _sc_dec_body,
        compiler_params=SCP,
        out_type=jax.ShapeDtypeStruct((NW * DPW,), jnp.float32),
        mesh=mesh,
        scratch_types=[
            pltpu.VMEM((DNCH, DC), jnp.int32),
            pltpu.VMEM((DNCH, DC), jnp.int32),
            pltpu.VMEM((DC, H), jnp.float32),
            pltpu.VMEM((DC, H), jnp.float32),
            pltpu.VMEM((DC, H), jnp.float32),
            pltpu.VMEM((DC, H), jnp.float32),
            pltpu.VMEM((H,), jnp.float32),
            pltpu.VMEM((16, 16), jnp.float32),
            pltpu.VMEM((DPW,), jnp.float32),
            pltpu.SemaphoreType.DMA,
            pltpu.SemaphoreType.DMA,
        ],
    )(p, q, ridx, cidx, w2)


def _pad_edges(e):
    npad = EPW - E // NW
    s = jnp.concatenate(
        [e[0].astype(jnp.int32).reshape(NW, E // NW),
         jnp.zeros((NW, npad), jnp.int32)], axis=1).reshape(NW, NCH, C)
    d = jnp.concatenate(
        [e[1].astype(jnp.int32).reshape(NW, E // NW),
         jnp.broadcast_to(N + jnp.arange(npad, dtype=jnp.int32),
                          (NW, npad))], axis=1).reshape(NW, NCH, C)
    return s, d


def _pad_a(a):
    return jnp.pad(a, (0, NP - N))


def kernel(x_sotu, x_taxon, params, node_id_sotu, node_id_taxon,
           edge_index_fwd, edge_index_rev, edge_label_index):
    p = params
    c1f, c1r, c2f, c2r = p["c1_fwd"], p["c1_rev"], p["c2_fwd"], p["c2_rev"]

    sf, df = _pad_edges(edge_index_fwd)
    sr, dr = _pad_edges(edge_index_rev)
    pad = NW * DPW - EL
    ridx = jnp.pad(edge_label_index[0].astype(jnp.int32), (0, pad)).reshape(NW, DNCH, DC)
    cidx = jnp.pad(edge_label_index[1].astype(jnp.int32), (0, pad)).reshape(NW, DNCH, DC)

    tab_s, a1_s, a2_s = _prep(x_sotu, p["sotu_emb"][node_id_sotu],
                              p["sotu_lin_W"], p["sotu_lin_b"].reshape(1, H),
                              c1f["W_src"], c1r["W_dst"],
                              c1f["att_src"].reshape(1, H),
                              c1r["att_dst"].reshape(1, H))
    tab_t, a1_t, a2_t = _prep(x_taxon, p["taxon_emb"][node_id_taxon],
                              p["taxon_lin_W"], p["taxon_lin_b"].reshape(1, H),
                              c1r["W_src"], c1f["W_dst"],
                              c1r["att_src"].reshape(1, H),
                              c1f["att_dst"].reshape(1, H))

    part_t, den_t = _sc_gat(tab_s, _pad_a(a1_s[:, 0]), _pad_a(a2_t[:, 0]), sf, df)
    part_s, den_s = _sc_gat(tab_t, _pad_a(a1_t[:, 0]), _pad_a(a2_s[:, 0]), sr, dr)

    zeros_h = jnp.zeros((1, H), jnp.float32)
    tab2_f, a1_s2, a2_s2 = _mid(part_s, den_s[:, :, None],
                                c1r["bias"].reshape(1, H),
                                c2f["W_src"], zeros_h, c2r["W_dst"],
                                c2f["att_src"].reshape(1, H),
                                c2r["att_dst"].reshape(1, H), relu=True)
    tab2_r, a1_t2, a2_t2 = _mid(part_t, den_t[:, :, None],
                                c1f["bias"].reshape(1, H),
                                c2r["W_src"], zeros_h, c2f["W_dst"],
                                c2r["att_src"].reshape(1, H),
                                c2f["att_dst"].reshape(1, H), relu=True)

    part2_t, den2_t = _sc_gat(tab2_f, _pad_a(a1_s2[:, 0]), _pad_a(a2_t2[:, 0]), sf, df)
    part2_s, den2_s = _sc_gat(tab2_r, _pad_a(a1_t2[:, 0]), _pad_a(a2_s2[:, 0]), sr, dr)

    zeros_hh = jnp.zeros((H, H), jnp.float32)
    pw, _, _ = _mid(part2_s, den2_s[:, :, None], c2r["bias"].reshape(1, H),
                    p["dec_W1"][:H], p["dec_b1"].reshape(1, H), zeros_hh,
                    zeros_h, zeros_h, relu=False)
    qw, _, _ = _mid(part2_t, den2_t[:, :, None], c2f["bias"].reshape(1, H),
                    p["dec_W1"][H:], zeros_h, zeros_hh,
                    zeros_h, zeros_h, relu=False)

    dec = _sc_dec(pw, qw, ridx, cidx, p["dec_W2"][:, 0])
    return dec[:EL] + p["dec_b2"][0]


# speedup vs baseline: 2.0759x; 2.0759x over previous
import functools

import jax
import jax.numpy as jnp
from jax import lax
from jax.experimental import pallas as pl
from jax.experimental.pallas import tpu as pltpu
from jax.experimental.pallas import tpu_sc as plsc

N = 10000
E = 320000
EL = 100000
H = 128
NW = 32
C = 80
EPW = E // NW
NCH = EPW // C
IB = 25
NB = NCH // IB
NP = 10240
ROWS_PW = NP // 16
DC = 80
DPW = 102400 // NW
DNCH = DPW // DC
DIB = 8
SCP = pltpu.CompilerParams(needs_layout_passes=False, use_tc_tiling_on_sc=False)


def _prep_body(x_ref, emb_ref, lw_ref, lb_ref, w_ref, w2_ref, atta_ref,
               attb_ref, tab_ref, a1_ref, a2_ref):
    h = x_ref[...] * lw_ref[...] + lb_ref[...] + emb_ref[...]
    tab = jnp.dot(h, w_ref[...], preferred_element_type=jnp.float32)
    hd = jnp.dot(h, w2_ref[...], preferred_element_type=jnp.float32)
    tab_ref[...] = tab
    a1_ref[...] = jnp.sum(tab * atta_ref[...], axis=1, keepdims=True)
    a2_ref[...] = jnp.sum(hd * attb_ref[...], axis=1, keepdims=True)


def _prep(x, emb, lin_w, lin_b, w, w2, atta, attb, block=2000):
    return pl.pallas_call(
        _prep_body,
        grid=(N // block,),
        in_specs=[
            pl.BlockSpec((block, 1), lambda i: (i, 0)),
            pl.BlockSpec((block, H), lambda i: (i, 0)),
            pl.BlockSpec((1, H), lambda i: (0, 0)),
            pl.BlockSpec((1, H), lambda i: (0, 0)),
            pl.BlockSpec((H, H), lambda i: (0, 0)),
            pl.BlockSpec((H, H), lambda i: (0, 0)),
            pl.BlockSpec((1, H), lambda i: (0, 0)),
            pl.BlockSpec((1, H), lambda i: (0, 0)),
        ],
        out_specs=[
            pl.BlockSpec((block, H), lambda i: (i, 0)),
            pl.BlockSpec((block, 1), lambda i: (i, 0)),
            pl.BlockSpec((block, 1), lambda i: (i, 0)),
        ],
        out_shape=[
            jax.ShapeDtypeStruct((N, H), jnp.float32),
            jax.ShapeDtypeStruct((N, 1), jnp.float32),
            jax.ShapeDtypeStruct((N, 1), jnp.float32),
        ],
    )(x, emb, lin_w, lin_b, w, w2, atta, attb)


def _mid_body(part_ref, den_ref, bias_ref, w_ref, ba_ref, w2_ref, atta_ref,
              attb_ref, tab_ref, a1_ref, a2_ref, *, relu):
    num = part_ref[0] + part_ref[1]
    den = den_ref[0] + den_ref[1]
    z = num / (den + 1e-16) + bias_ref[...]
    if relu:
        z = jnp.maximum(z, 0.0)
    tab = jnp.dot(z, w_ref[...],
                  preferred_element_type=jnp.float32) + ba_ref[...]
    hd = jnp.dot(z, w2_ref[...], preferred_element_type=jnp.float32)
    tab_ref[...] = tab
    a1_ref[...] = jnp.sum(tab * atta_ref[...], axis=1, keepdims=True)
    a2_ref[...] = jnp.sum(hd * attb_ref[...], axis=1, keepdims=True)


def _mid(part, den, bias, w, ba, w2, atta, attb, relu, block=2000):
    wout = w.shape[1]
    return pl.pallas_call(
        functools.partial(_mid_body, relu=relu),
        grid=(N // block,),
        in_specs=[
            pl.BlockSpec((2, block, H), lambda i: (0, i, 0)),
            pl.BlockSpec((2, block, 1), lambda i: (0, i, 0)),
            pl.BlockSpec((1, H), lambda i: (0, 0)),
            pl.BlockSpec((H, wout), lambda i: (0, 0)),
            pl.BlockSpec((1, wout), lambda i: (0, 0)),
            pl.BlockSpec((H, H), lambda i: (0, 0)),
            pl.BlockSpec((1, wout), lambda i: (0, 0)),
            pl.BlockSpec((1, H), lambda i: (0, 0)),
        ],
        out_specs=[
            pl.BlockSpec((block, wout), lambda i: (i, 0)),
            pl.BlockSpec((block, 1), lambda i: (i, 0)),
            pl.BlockSpec((block, 1), lambda i: (i, 0)),
        ],
        out_shape=[
            jax.ShapeDtypeStruct((N, wout), jnp.float32),
            jax.ShapeDtypeStruct((N, 1), jnp.float32),
            jax.ShapeDtypeStruct((N, 1), jnp.float32),
        ],
    )(part, den, bias, w, ba, w2, atta, attb)


def _sc_gat_body(tab_hbm, asrc_hbm, adst_hbm, srcs_hbm, dsts_hbm,
                 out_hbm, dout_hbm,
                 srcs_v, dsts_v, asrc_v, adst_v, r0, r1, w0, w1,
                 acc_sh, den_sh, sg0, sg1, ss0, ss1):
    cid = lax.axis_index("c")
    sid = lax.axis_index("s")
    wid = cid * 16 + sid
    rows = (r0, r1)
    wcs = (w0, w1)
    sgs = (sg0, sg1)
    sss = (ss0, ss1)

    pltpu.sync_copy(asrc_hbm, asrc_v)
    pltpu.sync_copy(adst_hbm, adst_v)

    def zrow(j, carry):
        for k in range(H // 16):
            r0[j, pl.ds(k * 16, 16)] = jnp.zeros((16,), jnp.float32)
        return carry

    lax.fori_loop(0, C, zrow, 0)
    for q in range(C // 16):
        w0[pl.ds(q * 16, 16)] = jnp.zeros((16,), jnp.float32)
    base = sid * ROWS_PW
    for i in range(ROWS_PW // C):
        pltpu.sync_copy(r0, acc_sh.at[pl.ds(base + i * C, C)])
        pltpu.sync_copy(w0, den_sh.at[pl.ds(base + i * C, C)])
    plsc.subcore_barrier()

    def calc_w(i, p):
        for q in range(C // 16):
            s16 = srcs_v[i, pl.ds(q * 16, 16)]
            d16 = dsts_v[i, pl.ds(q * 16, 16)]
            t = plsc.load_gather(asrc_v, [s16]) + plsc.load_gather(adst_v, [d16])
            t = jnp.maximum(t, 0.2 * t)
            wcs[p][pl.ds(q * 16, 16)] = jnp.exp(t)

    def scale(p):
        buf = rows[p]
        wc = wcs[p]

        def srow(j4, c2):
            for u in range(4):
                j = j4 * 4 + u
                w = plsc.load_gather(wc, [jnp.full((16,), j, jnp.int32)])
                for k in range(H // 16):
                    buf[j, pl.ds(k * 16, 16)] = buf[j, pl.ds(k * 16, 16)] * w
            return c2

        lax.fori_loop(0, C // 4, srow, 0)

    def blk(b, carry):
        pltpu.sync_copy(srcs_hbm.at[wid, pl.ds(b * IB, IB)], srcs_v)
        pltpu.sync_copy(dsts_hbm.at[wid, pl.ds(b * IB, IB)], dsts_v)
        gets = {}
        puts = {}
        gets[0] = pltpu.async_copy(tab_hbm.at[srcs_v.at[0]], r0, sg0)
        for i in range(IB):
            p = i % 2
            if i + 1 < IB:
                if i + 1 >= 2:
                    for d in puts[i - 1]:
                        d.wait()
                gets[i + 1] = pltpu.async_copy(
                    tab_hbm.at[srcs_v.at[i + 1]], rows[1 - p], sgs[1 - p])
            calc_w(i, p)
            gets[i].wait()
            scale(p)
            puts[i] = (
                pltpu.async_copy(rows[p], acc_sh.at[dsts_v.at[i]], sss[p],
                                 add=True),
                pltpu.async_copy(wcs[p], den_sh.at[dsts_v.at[i]], sss[p],
                                 add=True),
            )
        for d in puts[IB - 2] + puts[IB - 1]:
            d.wait()
        return carry

    lax.fori_loop(0, NB, blk, 0)
    plsc.subcore_barrier()
    pltpu.sync_copy(acc_sh.at[pl.ds(base, ROWS_PW)],
                    out_hbm.at[cid, pl.ds(base, ROWS_PW)])
    pltpu.sync_copy(den_sh.at[pl.ds(base, ROWS_PW)],
                    dout_hbm.at[cid, pl.ds(base, ROWS_PW)])


def _sc_gat(tab, a_src, a_dst, srcs, dsts):
    mesh = plsc.VectorSubcoreMesh(core_axis_name="c", subcore_axis_name="s")
    return pl.kernel(
        _sc_gat_body,
        compiler_params=SCP,
        out_type=[
            jax.ShapeDtypeStruct((2, NP, H), jnp.float32),
            jax.ShapeDtypeStruct((2, NP), jnp.float32),
        ],
        mesh=mesh,
        scratch_types=[
            pltpu.VMEM((IB, C), jnp.int32),
            pltpu.VMEM((IB, C), jnp.int32),
            pltpu.VMEM((N,), jnp.float32),
            pltpu.VMEM((N,), jnp.float32),
            pltpu.VMEM((C, H), jnp.float32),
            pltpu.VMEM((C, H), jnp.float32),
            pltpu.VMEM((C,), jnp.float32),
            pltpu.VMEM((C,), jnp.float32),
            pltpu.VMEM_SHARED((NP, H), jnp.float32),
            pltpu.VMEM_SHARED((NP,), jnp.float32),
            pltpu.SemaphoreType.DMA,
            pltpu.SemaphoreType.DMA,
            pltpu.SemaphoreType.DMA,
            pltpu.SemaphoreType.DMA,
        ],
    )(tab, a_src, a_dst, srcs, dsts)


def _sc_dec_body(p_hbm, q_hbm, ridx_hbm, cidx_hbm, w2_hbm, out_hbm,
                 ridx_v, cidx_v, p0, p1, q0, q1, w2_v, tbuf, obuf,
                 sg0, sg1):
    cid = lax.axis_index("c")
    sid = lax.axis_index("s")
    wid = cid * 16 + sid
    pbufs = (p0, p1)
    qbufs = (q0, q1)
    sgs = (sg0, sg1)

    pltpu.sync_copy(ridx_hbm.at[wid], ridx_v)
    pltpu.sync_copy(cidx_hbm.at[wid], cidx_v)
    pltpu.sync_copy(w2_hbm, w2_v)
    lane = jnp.arange(16, dtype=jnp.int32)
    w2s = [w2_v[pl.ds(k * 16, 16)] for k in range(H // 16)]

    def compute(g, p):
        pb = pbufs[p]
        qb = qbufs[p]

        def grp(j16, c2):
            for jj in range(16):
                acc = jnp.zeros((16,), jnp.float32)
                for k in range(H // 16):
                    pv = pb[j16 * 16 + jj, pl.ds(k * 16, 16)]
                    qv = qb[j16 * 16 + jj, pl.ds(k * 16, 16)]
                    acc = acc + jnp.maximum(pv + qv, 0.0) * w2s[k]
                tbuf[jj, :] = acc
            res = jnp.zeros((16,), jnp.float32)
            for k in range(16):
                res = res + plsc.load_gather(
                    tbuf, [lane, jnp.full((16,), k, jnp.int32)])
            obuf[pl.ds(g * DC + j16 * 16, 16)] = res
            return c2

        lax.fori_loop(0, DC // 16, grp, 0)

    def blk(b, carry):
        g0 = b * DIB
        gets = {}
        gets[0] = (
            pltpu.async_copy(p_hbm.at[ridx_v.at[g0]], p0, sg0),
            pltpu.async_copy(q_hbm.at[cidx_v.at[g0]], q0, sg0),
        )
        for i in range(DIB):
            p = i % 2
            if i + 1 < DIB:
                gets[i + 1] = (
                    pltpu.async_copy(p_hbm.at[ridx_v.at[g0 + i + 1]],
                                     pbufs[1 - p], sgs[1 - p]),
                    pltpu.async_copy(q_hbm.at[cidx_v.at[g0 + i + 1]],
                                     qbufs[1 - p], sgs[1 - p]),
                )
            for d in gets[i]:
                d.wait()
            compute(g0 + i, p)
        return carry

    lax.fori_loop(0, DNCH // DIB, blk, 0)
    pltpu.sync_copy(obuf, out_hbm.at[pl.ds(wid * DPW, DPW)])


def _sc_dec(p, q, ridx, cidx, w2):
    mesh = plsc.VectorSubcoreMesh(core_axis_name="c", subcore_axis_name="s")
    return pl.kernel(
        _sc_dec_body,
        compiler_params=SCP,
        out_type=jax.ShapeDtypeStruct((NW * DPW,), jnp.float32),
        mesh=mesh,
        scratch_types=[
            pltpu.VMEM((DNCH, DC), jnp.int32),
            pltpu.VMEM((DNCH, DC), jnp.int32),
            pltpu.VMEM((DC, H), jnp.float32),
            pltpu.VMEM((DC, H), jnp.float32),
            pltpu.VMEM((DC, H), jnp.float32),
            pltpu.VMEM((DC, H), jnp.float32),
            pltpu.VMEM((H,), jnp.float32),
            pltpu.VMEM((16, 16), jnp.float32),
            pltpu.VMEM((DPW,), jnp.float32),
            pltpu.SemaphoreType.DMA,
            pltpu.SemaphoreType.DMA,
        ],
    )(p, q, ridx, cidx, w2)


def _pad_edges(e):
    return (e[0].astype(jnp.int32).reshape(NW, NCH, C),
            e[1].astype(jnp.int32).reshape(NW, NCH, C))


def kernel(x_sotu, x_taxon, params, node_id_sotu, node_id_taxon,
           edge_index_fwd, edge_index_rev, edge_label_index):
    p = params
    c1f, c1r, c2f, c2r = p["c1_fwd"], p["c1_rev"], p["c2_fwd"], p["c2_rev"]

    sf, df = _pad_edges(edge_index_fwd)
    sr, dr = _pad_edges(edge_index_rev)
    pad = NW * DPW - EL
    ridx = jnp.pad(edge_label_index[0].astype(jnp.int32), (0, pad)).reshape(NW, DNCH, DC)
    cidx = jnp.pad(edge_label_index[1].astype(jnp.int32), (0, pad)).reshape(NW, DNCH, DC)

    tab_s, a1_s, a2_s = _prep(x_sotu, p["sotu_emb"][node_id_sotu],
                              p["sotu_lin_W"], p["sotu_lin_b"].reshape(1, H),
                              c1f["W_src"], c1r["W_dst"],
                              c1f["att_src"].reshape(1, H),
                              c1r["att_dst"].reshape(1, H))
    tab_t, a1_t, a2_t = _prep(x_taxon, p["taxon_emb"][node_id_taxon],
                              p["taxon_lin_W"], p["taxon_lin_b"].reshape(1, H),
                              c1r["W_src"], c1f["W_dst"],
                              c1r["att_src"].reshape(1, H),
                              c1f["att_dst"].reshape(1, H))

    part_t, den_t = _sc_gat(tab_s, (a1_s[:, 0]), (a2_t[:, 0]), sf, df)
    part_s, den_s = _sc_gat(tab_t, (a1_t[:, 0]), (a2_s[:, 0]), sr, dr)

    zeros_h = jnp.zeros((1, H), jnp.float32)
    tab2_f, a1_s2, a2_s2 = _mid(part_s, den_s[:, :, None],
                                c1r["bias"].reshape(1, H),
                                c2f["W_src"], zeros_h, c2r["W_dst"],
                                c2f["att_src"].reshape(1, H),
                                c2r["att_dst"].reshape(1, H), relu=True)
    tab2_r, a1_t2, a2_t2 = _mid(part_t, den_t[:, :, None],
                                c1f["bias"].reshape(1, H),
                                c2r["W_src"], zeros_h, c2f["W_dst"],
                                c2r["att_src"].reshape(1, H),
                                c2f["att_dst"].reshape(1, H), relu=True)

    part2_t, den2_t = _sc_gat(tab2_f, (a1_s2[:, 0]), (a2_t2[:, 0]), sf, df)
    part2_s, den2_s = _sc_gat(tab2_r, (a1_t2[:, 0]), (a2_s2[:, 0]), sr, dr)

    zeros_hh = jnp.zeros((H, H), jnp.float32)
    pw, _, _ = _mid(part2_s, den2_s[:, :, None], c2r["bias"].reshape(1, H),
                    p["dec_W1"][:H], p["dec_b1"].reshape(1, H), zeros_hh,
                    zeros_h, zeros_h, relu=False)
    qw, _, _ = _mid(part2_t, den2_t[:, :, None], c2f["bias"].reshape(1, H),
                    p["dec_W1"][H:], zeros_h, zeros_hh,
                    zeros_h, zeros_h, relu=False)

    dec = _sc_dec(pw, qw, ridx, cidx, p["dec_W2"][:, 0])
    return dec[:EL] + p["dec_b2"][0]
